# baseline (speedup 1.0000x reference)
import functools

import jax
import jax.numpy as jnp
from jax import lax
from jax.experimental import pallas as pl
from jax.experimental.pallas import tpu as pltpu
from jax.experimental.pallas import tpu_sc as plsc

N = 2048
E = 8
T = 256
NT = N // T + E
N_PAD = NT * T
D = 256
RB = 256


def _relu(x):
    return jnp.maximum(x, 0.0)


def _routing_body(iv_hbm, pos0_ref, pos1_ref, sched_ref, oh_v, soh):
    coh = pltpu.make_async_copy(iv_hbm.at[:, pl.ds(512, 16)], oh_v, soh)
    coh.start()
    coh.wait()
    ri = lax.broadcasted_iota(jnp.int32, (RB, RB), 0)
    ci = lax.broadcasted_iota(jnp.int32, (RB, RB), 1)
    slt = jnp.where(ri > ci, 1.0, 0.0)
    carry = jnp.zeros((1, 16), jnp.float32)
    ranks = []
    ohs = []
    for blk in range(N // RB):
        o = oh_v[blk * RB:(blk + 1) * RB, :]
        csum_excl = jnp.dot(slt, o, preferred_element_type=jnp.float32) + carry
        carry = carry + jnp.sum(o, axis=0, keepdims=True)
        ranks.append(csum_excl * o)
        ohs.append(o)
    counts = carry
    padded = jnp.floor((counts + (T - 1)) * (1.0 / T)) * T
    gi = lax.broadcasted_iota(jnp.int32, (16, 16), 0)
    gj = lax.broadcasted_iota(jnp.int32, (16, 16), 1)
    same_group = (gi < 8) == (gj < 8)
    slt16 = jnp.where((gi < gj) & same_group, 1.0, 0.0)
    off = jnp.dot(padded, slt16, preferred_element_type=jnp.float32)
    for blk in range(N // RB):
        o = ohs[blk]
        slot = ranks[blk] + o * off
        pos0_ref[blk * RB:(blk + 1) * RB] = jnp.sum(
            slot[:, 0:8], axis=1).astype(jnp.int32)
        pos1_ref[blk * RB:(blk + 1) * RB] = jnp.sum(
            slot[:, 8:16], axis=1).astype(jnp.int32)
    starts = (lax.broadcasted_iota(jnp.int32, (NT, 16), 0) * T).astype(jnp.float32)
    lane = lax.broadcasted_iota(jnp.int32, (NT, 16), 1)
    offb = jnp.broadcast_to(off, (NT, 16))
    le = jnp.where(offb <= starts, 1, 0)
    texp0 = jnp.sum(jnp.where(lane < 8, le, 0), axis=1, keepdims=True) - 1
    texp1 = jnp.sum(jnp.where(lane >= 8, le, 0), axis=1, keepdims=True) - 1
    total = off + padded
    tot0 = jnp.sum(jnp.where(lane[0:1] == 7, jnp.broadcast_to(total, (1, 16)), 0.0),
                   axis=1, keepdims=True)
    tot1 = jnp.sum(jnp.where(lane[0:1] == 15, jnp.broadcast_to(total, (1, 16)), 0.0),
                   axis=1, keepdims=True)
    tval0 = jnp.where(starts[:, 0:1] < tot0, 1, 0)
    tval1 = jnp.where(starts[:, 0:1] < tot1, 1, 0)
    sched_ref[...] = jnp.concatenate(
        [texp0, tval0, texp1, tval1, jnp.zeros((NT, 4), jnp.int32)], axis=1)


def _routing(input_val):
    return pl.pallas_call(
        _routing_body,
        in_specs=[pl.BlockSpec(memory_space=pl.ANY)],
        out_specs=[pl.BlockSpec((N,), lambda: (0,)),
                   pl.BlockSpec((N,), lambda: (0,)),
                   pl.BlockSpec((NT, 8), lambda: (0, 0))],
        out_shape=[jax.ShapeDtypeStruct((N,), jnp.int32),
                   jax.ShapeDtypeStruct((N,), jnp.int32),
                   jax.ShapeDtypeStruct((NT, 8), jnp.int32)],
        scratch_shapes=[pltpu.VMEM((N, 16), jnp.float32),
                        pltpu.SemaphoreType.DMA],
    )(input_val)


def _row_select(b_ref, te):
    b = b_ref[...]
    rows = lax.broadcasted_iota(jnp.int32, b.shape, 0)
    return jnp.sum(jnp.where(rows == te, b, 0.0), axis=0, keepdims=True)


def _mlp0_body(sched_ref, x_hbm, wa_hbm, ba_hbm, wb_hbm, bb_hbm, wc_hbm, bc_hbm,
               out_hbm, x_v, wa_v, ba_v, wb_v, bb_v, wc_v, bc_v, h1_v, h2_v, o_v,
               sx, sa, sba, sb, sbb, sc, sbc, so):
    cxs = [pltpu.make_async_copy(x_hbm.at[pl.ds(i * T, T)],
                                 x_v.at[pl.ds(i * T, T)], sx) for i in range(NT)]
    ca = pltpu.make_async_copy(wa_hbm, wa_v, sa)
    cb = pltpu.make_async_copy(wb_hbm, wb_v, sb)
    cc = pltpu.make_async_copy(wc_hbm, wc_v, sc)
    cba = pltpu.make_async_copy(ba_hbm, ba_v, sba)
    cbb = pltpu.make_async_copy(bb_hbm, bb_v, sbb)
    cbc = pltpu.make_async_copy(bc_hbm, bc_v, sbc)
    for c in (cba, cbb, cbc, *cxs, ca, cb, cc):
        c.start()
    ca.wait(); cba.wait()
    for i in range(NT):
        cxs[i].wait()

        @pl.when(sched_ref[i, 1] > 0)
        def _():
            te = sched_ref[i, 0]
            h1_v[i * T:(i + 1) * T, :] = _relu(
                jnp.dot(x_v[i * T:(i + 1) * T, :], wa_v[te],
                        preferred_element_type=jnp.float32) + _row_select(ba_v, te))
    cb.wait(); cbb.wait()
    for i in range(NT):
        @pl.when(sched_ref[i, 1] > 0)
        def _():
            te = sched_ref[i, 0]
            h2_v[i * T:(i + 1) * T, :] = _relu(
                jnp.dot(h1_v[i * T:(i + 1) * T, :], wb_v[te],
                        preferred_element_type=jnp.float32) + _row_select(bb_v, te))
    cc.wait(); cbc.wait()
    for i in range(NT):
        @pl.when(sched_ref[i, 1] > 0)
        def _():
            te = sched_ref[i, 0]
            o_v[i * T:(i + 1) * T, :] = _relu(
                jnp.dot(h2_v[i * T:(i + 1) * T, :], wc_v[te],
                        preferred_element_type=jnp.float32) + _row_select(bc_v, te))
            pltpu.make_async_copy(o_v.at[pl.ds(i * T, T)],
                                  out_hbm.at[pl.ds(i * T, T)], so).start()
    for i in range(NT):
        @pl.when(sched_ref[i, 1] > 0)
        def _():
            pltpu.make_async_copy(o_v.at[pl.ds(i * T, T)],
                                  out_hbm.at[pl.ds(i * T, T)], so).wait()


def _tile_mlp0(sched, x_s, W0a, b0a, W0b, b0b, W0c, b0c):
    spec = pltpu.PrefetchScalarGridSpec(
        num_scalar_prefetch=1,
        grid=(1,),
        in_specs=[pl.BlockSpec(memory_space=pl.ANY)] * 7,
        out_specs=pl.BlockSpec(memory_space=pl.ANY),
        scratch_shapes=(
            [pltpu.VMEM((N_PAD, 256), jnp.float32),
             pltpu.VMEM((E, 256, 512), jnp.float32),
             pltpu.VMEM((E, 512), jnp.float32),
             pltpu.VMEM((E, 512, 512), jnp.float32),
             pltpu.VMEM((E, 512), jnp.float32),
             pltpu.VMEM((E, 512, 256), jnp.float32),
             pltpu.VMEM((E, 256), jnp.float32),
             pltpu.VMEM((N_PAD, 512), jnp.float32),
             pltpu.VMEM((N_PAD, 512), jnp.float32),
             pltpu.VMEM((N_PAD, 256), jnp.float32)]
            + [pltpu.SemaphoreType.DMA] * 8
        ),
    )
    return pl.pallas_call(
        _mlp0_body,
        grid_spec=spec,
        out_shape=jax.ShapeDtypeStruct((N_PAD, D), jnp.float32),
    )(sched, x_s, W0a, b0a, W0b, b0b, W0c, b0c)


def _mlp1_body(sched_ref, x_hbm, prev_hbm, wp_hbm, bp_hbm, wi_hbm, bi_hbm,
               wo_hbm, bo_hbm, out_hbm, x_v, prev_v, wp_v, bp_v, wit_v, wib_v,
               bi_v, wo_v, bo_v, p_v, h_v, o_v,
               sx, sp, swp, sbp, swit, swib, sbi, swo, sbo, so):
    cxs = [pltpu.make_async_copy(x_hbm.at[pl.ds(i * T, T)],
                                 x_v.at[pl.ds(i * T, T)], sx) for i in range(NT)]
    cp = pltpu.make_async_copy(prev_hbm, prev_v, sp)
    cwp = pltpu.make_async_copy(wp_hbm, wp_v, swp)
    cwit = pltpu.make_async_copy(wi_hbm.at[:, pl.ds(0, 256), :], wit_v, swit)
    cwib = pltpu.make_async_copy(wi_hbm.at[:, pl.ds(256, 512), :], wib_v, swib)
    cwo = pltpu.make_async_copy(wo_hbm, wo_v, swo)
    cbp = pltpu.make_async_copy(bp_hbm, bp_v, sbp)
    cbi = pltpu.make_async_copy(bi_hbm, bi_v, sbi)
    cbo = pltpu.make_async_copy(bo_hbm, bo_v, sbo)
    for c in (cbp, cbi, cbo, *cxs, cwp, cp, cwit, cwib, cwo):
        c.start()
    cwp.wait(); cbp.wait()
    for i in range(NT):
        cxs[i].wait()

        @pl.when(sched_ref[i, 3] > 0)
        def _():
            te = sched_ref[i, 2]
            p_v[i * T:(i + 1) * T, :] = _relu(
                jnp.dot(x_v[i * T:(i + 1) * T, :], wp_v[te],
                        preferred_element_type=jnp.float32) + _row_select(bp_v, te))
    cp.wait(); cwit.wait()
    for i in range(NT):
        @pl.when(sched_ref[i, 3] > 0)
        def _():
            te = sched_ref[i, 2]
            h_v[i * T:(i + 1) * T, :] = jnp.dot(
                prev_v[i * T:(i + 1) * T, :], wit_v[te],
                preferred_element_type=jnp.float32)
    cwib.wait(); cbi.wait()
    for i in range(NT):
        @pl.when(sched_ref[i, 3] > 0)
        def _():
            te = sched_ref[i, 2]
            h_v[i * T:(i + 1) * T, :] = _relu(
                h_v[i * T:(i + 1) * T, :]
                + jnp.dot(p_v[i * T:(i + 1) * T, :], wib_v[te],
                          preferred_element_type=jnp.float32)
                + _row_select(bi_v, te))
    cwo.wait(); cbo.wait()
    for i in range(NT):
        @pl.when(sched_ref[i, 3] > 0)
        def _():
            te = sched_ref[i, 2]
            o_v[i * T:(i + 1) * T, :] = (
                jnp.dot(h_v[i * T:(i + 1) * T, :], wo_v[te],
                        preferred_element_type=jnp.float32) + _row_select(bo_v, te))
            pltpu.make_async_copy(o_v.at[pl.ds(i * T, T)],
                                  out_hbm.at[pl.ds(i * T, T)], so).start()
    for i in range(NT):
        @pl.when(sched_ref[i, 3] > 0)
        def _():
            pltpu.make_async_copy(o_v.at[pl.ds(i * T, T)],
                                  out_hbm.at[pl.ds(i * T, T)], so).wait()


def _tile_mlp1(sched, x_s, prev_s, W1pre, b1pre, W1int, b1int, W1out, b1out):
    spec = pltpu.PrefetchScalarGridSpec(
        num_scalar_prefetch=1,
        grid=(1,),
        in_specs=[pl.BlockSpec(memory_space=pl.ANY)] * 8,
        out_specs=pl.BlockSpec(memory_space=pl.ANY),
        scratch_shapes=(
            [pltpu.VMEM((N_PAD, 256), jnp.float32),
             pltpu.VMEM((N_PAD, 256), jnp.float32),
             pltpu.VMEM((E, 256, 512), jnp.float32),
             pltpu.VMEM((E, 512), jnp.float32),
             pltpu.VMEM((E, 256, 512), jnp.float32),
             pltpu.VMEM((E, 512, 512), jnp.float32),
             pltpu.VMEM((E, 512), jnp.float32),
             pltpu.VMEM((E, 512, 256), jnp.float32),
             pltpu.VMEM((E, 256), jnp.float32),
             pltpu.VMEM((N_PAD, 512), jnp.float32),
             pltpu.VMEM((N_PAD, 512), jnp.float32),
             pltpu.VMEM((N_PAD, 256), jnp.float32)]
            + [pltpu.SemaphoreType.DMA] * 10
        ),
    )
    return pl.pallas_call(
        _mlp1_body,
        grid_spec=spec,
        out_shape=jax.ShapeDtypeStruct((N_PAD, D), jnp.float32),
    )(sched, x_s, prev_s, W1pre, b1pre, W1int, b1int, W1out, b1out)


def _make_sc_kernels():
    info = plsc.get_sparse_core_info()
    nc, ns = info.num_cores, info.num_subcores
    nw = nc * ns
    tok_w = N // nw
    mesh = plsc.VectorSubcoreMesh(core_axis_name="c", subcore_axis_name="s")

    def _wid():
        return lax.axis_index("s") * nc + lax.axis_index("c")

    @functools.partial(
        pl.kernel, mesh=mesh,
        out_type=[jax.ShapeDtypeStruct((N_PAD, D), jnp.float32),
                  jax.ShapeDtypeStruct((N_PAD, D), jnp.float32)],
        scratch_types=[
            pltpu.VMEM((tok_w,), jnp.int32), pltpu.VMEM((tok_w,), jnp.int32),
            pltpu.VMEM((tok_w, D), jnp.float32), pltpu.VMEM((tok_w, D), jnp.float32),
            pltpu.SemaphoreType.DMA, pltpu.SemaphoreType.DMA,
            pltpu.SemaphoreType.DMA, pltpu.SemaphoreType.DMA,
        ],
    )
    def scatter_in(x0_hbm, x1_hbm, pos0_hbm, pos1_hbm, x0s_hbm, x1s_hbm,
                   idx0_v, idx1_v, r0_v, r1_v, s0, s1, s2, s3):
        base = _wid() * tok_w
        c0 = pltpu.async_copy(x0_hbm.at[pl.ds(base, tok_w)], r0_v, s0)
        c1 = pltpu.async_copy(x1_hbm.at[pl.ds(base, tok_w)], r1_v, s1)
        c2 = pltpu.async_copy(pos0_hbm.at[pl.ds(base, tok_w)], idx0_v, s2)
        c3 = pltpu.async_copy(pos1_hbm.at[pl.ds(base, tok_w)], idx1_v, s3)
        c0.wait()
        c2.wait()
        c4 = pltpu.async_copy(r0_v, x0s_hbm.at[idx0_v], s0)
        c1.wait()
        c3.wait()
        c5 = pltpu.async_copy(r1_v, x1s_hbm.at[idx1_v], s1)
        c4.wait()
        c5.wait()

    @functools.partial(
        pl.kernel, mesh=mesh,
        out_type=jax.ShapeDtypeStruct((N_PAD, D), jnp.float32),
        scratch_types=[
            pltpu.VMEM((tok_w,), jnp.int32), pltpu.VMEM((tok_w,), jnp.int32),
            pltpu.VMEM((tok_w, D), jnp.float32),
            pltpu.SemaphoreType.DMA, pltpu.SemaphoreType.DMA,
        ],
    )
    def permute(h0s_hbm, pos0_hbm, pos1_hbm, prevs_hbm, idx0_v, idx1_v, rows_v, s0, s1):
        base = _wid() * tok_w
        c0 = pltpu.async_copy(pos0_hbm.at[pl.ds(base, tok_w)], idx0_v, s0)
        c1 = pltpu.async_copy(pos1_hbm.at[pl.ds(base, tok_w)], idx1_v, s1)
        c0.wait()
        pltpu.async_copy(h0s_hbm.at[idx0_v], rows_v, s0).wait()
        c1.wait()
        pltpu.async_copy(rows_v, prevs_hbm.at[idx1_v], s1).wait()

    @functools.partial(
        pl.kernel, mesh=mesh,
        out_type=jax.ShapeDtypeStruct((N, D), jnp.float32),
        scratch_types=[
            pltpu.VMEM((tok_w,), jnp.int32),
            pltpu.VMEM((tok_w, D), jnp.float32),
            pltpu.SemaphoreType.DMA,
        ],
    )
    def gather_out(o1s_hbm, pos1_hbm, out_hbm, idx_v, rows_v, sem):
        base = _wid() * tok_w
        pltpu.sync_copy(pos1_hbm.at[pl.ds(base, tok_w)], idx_v)
        pltpu.async_copy(o1s_hbm.at[idx_v], rows_v, sem).wait()
        pltpu.sync_copy(rows_v, out_hbm.at[pl.ds(base, tok_w)])

    return scatter_in, permute, gather_out


def kernel(input_val, W0a, b0a, W0b, b0b, W0c, b0c, W1pre, b1pre, W1int, b1int, W1out, b1out):
    pos0, pos1, sched = _routing(input_val)
    x0 = input_val[:, 0:256]
    x1 = input_val[:, 256:512]

    scatter_in, permute, gather_out = _make_sc_kernels()

    x0_s, x1_s = scatter_in(x0, x1, pos0, pos1)
    h0_s = _tile_mlp0(sched, x0_s, W0a, b0a, W0b, b0b, W0c, b0c)
    prev_s = permute(h0_s, pos0, pos1)
    o1_s = _tile_mlp1(sched, x1_s, prev_s, W1pre, b1pre, W1int, b1int, W1out, b1out)
    return gather_out(o1_s, pos1)


# speedup vs baseline: 1.0086x; 1.0086x over previous
import functools

import jax
import jax.numpy as jnp
from jax import lax
from jax.experimental import pallas as pl
from jax.experimental.pallas import tpu as pltpu
from jax.experimental.pallas import tpu_sc as plsc

N = 2048
E = 8
T = 256
NT = N // T + E
N_PAD = NT * T
D = 256
RB = 256


def _relu(x):
    return jnp.maximum(x, 0.0)


def _routing_body(iv_hbm, pos0_ref, pos1_ref, sched_ref, oh_v, soh):
    coh = pltpu.make_async_copy(iv_hbm, oh_v, soh)
    coh.start()
    coh.wait()
    ri = lax.broadcasted_iota(jnp.int32, (RB, RB), 0)
    ci = lax.broadcasted_iota(jnp.int32, (RB, RB), 1)
    slt = jnp.where(ri > ci, 1.0, 0.0)
    carry = jnp.zeros((1, 16), jnp.float32)
    ranks = []
    ohs = []
    for blk in range(N // RB):
        o = oh_v[blk * RB:(blk + 1) * RB, :]
        csum_excl = jnp.dot(slt, o, preferred_element_type=jnp.float32) + carry
        carry = carry + jnp.sum(o, axis=0, keepdims=True)
        ranks.append(csum_excl * o)
        ohs.append(o)
    counts = carry
    padded = jnp.floor((counts + (T - 1)) * (1.0 / T)) * T
    gi = lax.broadcasted_iota(jnp.int32, (16, 16), 0)
    gj = lax.broadcasted_iota(jnp.int32, (16, 16), 1)
    same_group = (gi < 8) == (gj < 8)
    slt16 = jnp.where((gi < gj) & same_group, 1.0, 0.0)
    off = jnp.dot(padded, slt16, preferred_element_type=jnp.float32)
    for blk in range(N // RB):
        o = ohs[blk]
        slot = ranks[blk] + o * off
        pos0_ref[blk * RB:(blk + 1) * RB] = jnp.sum(
            slot[:, 0:8], axis=1).astype(jnp.int32)
        pos1_ref[blk * RB:(blk + 1) * RB] = jnp.sum(
            slot[:, 8:16], axis=1).astype(jnp.int32)
    starts = (lax.broadcasted_iota(jnp.int32, (NT, 16), 0) * T).astype(jnp.float32)
    lane = lax.broadcasted_iota(jnp.int32, (NT, 16), 1)
    offb = jnp.broadcast_to(off, (NT, 16))
    le = jnp.where(offb <= starts, 1, 0)
    texp0 = jnp.sum(jnp.where(lane < 8, le, 0), axis=1, keepdims=True) - 1
    texp1 = jnp.sum(jnp.where(lane >= 8, le, 0), axis=1, keepdims=True) - 1
    total = off + padded
    tot0 = jnp.sum(jnp.where(lane[0:1] == 7, jnp.broadcast_to(total, (1, 16)), 0.0),
                   axis=1, keepdims=True)
    tot1 = jnp.sum(jnp.where(lane[0:1] == 15, jnp.broadcast_to(total, (1, 16)), 0.0),
                   axis=1, keepdims=True)
    tval0 = jnp.where(starts[:, 0:1] < tot0, 1, 0)
    tval1 = jnp.where(starts[:, 0:1] < tot1, 1, 0)
    sched_ref[...] = jnp.concatenate(
        [texp0, tval0, texp1, tval1, jnp.zeros((NT, 4), jnp.int32)], axis=1)


def _routing(input_val):
    return pl.pallas_call(
        _routing_body,
        in_specs=[pl.BlockSpec(memory_space=pl.ANY)],
        out_specs=[pl.BlockSpec((N,), lambda: (0,)),
                   pl.BlockSpec((N,), lambda: (0,)),
                   pl.BlockSpec((NT, 8), lambda: (0, 0))],
        out_shape=[jax.ShapeDtypeStruct((N,), jnp.int32),
                   jax.ShapeDtypeStruct((N,), jnp.int32),
                   jax.ShapeDtypeStruct((NT, 8), jnp.int32)],
        scratch_shapes=[pltpu.VMEM((N, 16), jnp.float32),
                        pltpu.SemaphoreType.DMA],
    )(input_val)


def _row_select(b_ref, te):
    b = b_ref[...]
    rows = lax.broadcasted_iota(jnp.int32, b.shape, 0)
    return jnp.sum(jnp.where(rows == te, b, 0.0), axis=0, keepdims=True)


def _mlp0_body(sched_ref, x_hbm, wa_hbm, ba_hbm, wb_hbm, bb_hbm, wc_hbm, bc_hbm,
               out_hbm, x_v, wa_v, ba_v, wb_v, bb_v, wc_v, bc_v, h1_v, h2_v, o_v,
               sx, sa, sba, sb, sbb, sc, sbc, so):
    cxs = [pltpu.make_async_copy(x_hbm.at[pl.ds(i * T, T)],
                                 x_v.at[pl.ds(i * T, T)], sx) for i in range(NT)]
    ca = pltpu.make_async_copy(wa_hbm, wa_v, sa)
    cb = pltpu.make_async_copy(wb_hbm, wb_v, sb)
    cc = pltpu.make_async_copy(wc_hbm, wc_v, sc)
    cba = pltpu.make_async_copy(ba_hbm, ba_v, sba)
    cbb = pltpu.make_async_copy(bb_hbm, bb_v, sbb)
    cbc = pltpu.make_async_copy(bc_hbm, bc_v, sbc)
    for c in (cba, cbb, cbc, *cxs, ca, cb, cc):
        c.start()
    ca.wait(); cba.wait()
    for i in range(NT):
        cxs[i].wait()

        @pl.when(sched_ref[i, 1] > 0)
        def _():
            te = sched_ref[i, 0]
            h1_v[i * T:(i + 1) * T, :] = _relu(
                jnp.dot(x_v[i * T:(i + 1) * T, :], wa_v[te],
                        preferred_element_type=jnp.float32) + _row_select(ba_v, te))
    cb.wait(); cbb.wait()
    for i in range(NT):
        @pl.when(sched_ref[i, 1] > 0)
        def _():
            te = sched_ref[i, 0]
            h2_v[i * T:(i + 1) * T, :] = _relu(
                jnp.dot(h1_v[i * T:(i + 1) * T, :], wb_v[te],
                        preferred_element_type=jnp.float32) + _row_select(bb_v, te))
    cc.wait(); cbc.wait()
    for i in range(NT):
        @pl.when(sched_ref[i, 1] > 0)
        def _():
            te = sched_ref[i, 0]
            o_v[i * T:(i + 1) * T, :] = _relu(
                jnp.dot(h2_v[i * T:(i + 1) * T, :], wc_v[te],
                        preferred_element_type=jnp.float32) + _row_select(bc_v, te))
            pltpu.make_async_copy(o_v.at[pl.ds(i * T, T)],
                                  out_hbm.at[pl.ds(i * T, T)], so).start()
    for i in range(NT):
        @pl.when(sched_ref[i, 1] > 0)
        def _():
            pltpu.make_async_copy(o_v.at[pl.ds(i * T, T)],
                                  out_hbm.at[pl.ds(i * T, T)], so).wait()


def _tile_mlp0(sched, x_s, W0a, b0a, W0b, b0b, W0c, b0c):
    spec = pltpu.PrefetchScalarGridSpec(
        num_scalar_prefetch=1,
        grid=(1,),
        in_specs=[pl.BlockSpec(memory_space=pl.ANY)] * 7,
        out_specs=pl.BlockSpec(memory_space=pl.ANY),
        scratch_shapes=(
            [pltpu.VMEM((N_PAD, 256), jnp.float32),
             pltpu.VMEM((E, 256, 512), jnp.float32),
             pltpu.VMEM((E, 512), jnp.float32),
             pltpu.VMEM((E, 512, 512), jnp.float32),
             pltpu.VMEM((E, 512), jnp.float32),
             pltpu.VMEM((E, 512, 256), jnp.float32),
             pltpu.VMEM((E, 256), jnp.float32),
             pltpu.VMEM((N_PAD, 512), jnp.float32),
             pltpu.VMEM((N_PAD, 512), jnp.float32),
             pltpu.VMEM((N_PAD, 256), jnp.float32)]
            + [pltpu.SemaphoreType.DMA] * 8
        ),
    )
    return pl.pallas_call(
        _mlp0_body,
        grid_spec=spec,
        out_shape=jax.ShapeDtypeStruct((N_PAD, D), jnp.float32),
    )(sched, x_s, W0a, b0a, W0b, b0b, W0c, b0c)


def _mlp1_body(sched_ref, x_hbm, prev_hbm, wp_hbm, bp_hbm, wi_hbm, bi_hbm,
               wo_hbm, bo_hbm, out_hbm, x_v, prev_v, wp_v, bp_v, wit_v, wib_v,
               bi_v, wo_v, bo_v, p_v, h_v, o_v,
               sx, sp, swp, sbp, swit, swib, sbi, swo, sbo, so):
    cxs = [pltpu.make_async_copy(x_hbm.at[pl.ds(i * T, T)],
                                 x_v.at[pl.ds(i * T, T)], sx) for i in range(NT)]
    cp = pltpu.make_async_copy(prev_hbm, prev_v, sp)
    cwp = pltpu.make_async_copy(wp_hbm, wp_v, swp)
    cwit = pltpu.make_async_copy(wi_hbm.at[:, pl.ds(0, 256), :], wit_v, swit)
    cwib = pltpu.make_async_copy(wi_hbm.at[:, pl.ds(256, 512), :], wib_v, swib)
    cwo = pltpu.make_async_copy(wo_hbm, wo_v, swo)
    cbp = pltpu.make_async_copy(bp_hbm, bp_v, sbp)
    cbi = pltpu.make_async_copy(bi_hbm, bi_v, sbi)
    cbo = pltpu.make_async_copy(bo_hbm, bo_v, sbo)
    for c in (cbp, cbi, cbo, *cxs, cwp, cp, cwit, cwib, cwo):
        c.start()
    cwp.wait(); cbp.wait()
    for i in range(NT):
        cxs[i].wait()

        @pl.when(sched_ref[i, 3] > 0)
        def _():
            te = sched_ref[i, 2]
            p_v[i * T:(i + 1) * T, :] = _relu(
                jnp.dot(x_v[i * T:(i + 1) * T, :], wp_v[te],
                        preferred_element_type=jnp.float32) + _row_select(bp_v, te))
    cp.wait(); cwit.wait()
    for i in range(NT):
        @pl.when(sched_ref[i, 3] > 0)
        def _():
            te = sched_ref[i, 2]
            h_v[i * T:(i + 1) * T, :] = jnp.dot(
                prev_v[i * T:(i + 1) * T, :], wit_v[te],
                preferred_element_type=jnp.float32)
    cwib.wait(); cbi.wait()
    for i in range(NT):
        @pl.when(sched_ref[i, 3] > 0)
        def _():
            te = sched_ref[i, 2]
            h_v[i * T:(i + 1) * T, :] = _relu(
                h_v[i * T:(i + 1) * T, :]
                + jnp.dot(p_v[i * T:(i + 1) * T, :], wib_v[te],
                          preferred_element_type=jnp.float32)
                + _row_select(bi_v, te))
    cwo.wait(); cbo.wait()
    for i in range(NT):
        @pl.when(sched_ref[i, 3] > 0)
        def _():
            te = sched_ref[i, 2]
            o_v[i * T:(i + 1) * T, :] = (
                jnp.dot(h_v[i * T:(i + 1) * T, :], wo_v[te],
                        preferred_element_type=jnp.float32) + _row_select(bo_v, te))
            pltpu.make_async_copy(o_v.at[pl.ds(i * T, T)],
                                  out_hbm.at[pl.ds(i * T, T)], so).start()
    for i in range(NT):
        @pl.when(sched_ref[i, 3] > 0)
        def _():
            pltpu.make_async_copy(o_v.at[pl.ds(i * T, T)],
                                  out_hbm.at[pl.ds(i * T, T)], so).wait()


def _tile_mlp1(sched, x_s, prev_s, W1pre, b1pre, W1int, b1int, W1out, b1out):
    spec = pltpu.PrefetchScalarGridSpec(
        num_scalar_prefetch=1,
        grid=(1,),
        in_specs=[pl.BlockSpec(memory_space=pl.ANY)] * 8,
        out_specs=pl.BlockSpec(memory_space=pl.ANY),
        scratch_shapes=(
            [pltpu.VMEM((N_PAD, 256), jnp.float32),
             pltpu.VMEM((N_PAD, 256), jnp.float32),
             pltpu.VMEM((E, 256, 512), jnp.float32),
             pltpu.VMEM((E, 512), jnp.float32),
             pltpu.VMEM((E, 256, 512), jnp.float32),
             pltpu.VMEM((E, 512, 512), jnp.float32),
             pltpu.VMEM((E, 512), jnp.float32),
             pltpu.VMEM((E, 512, 256), jnp.float32),
             pltpu.VMEM((E, 256), jnp.float32),
             pltpu.VMEM((N_PAD, 512), jnp.float32),
             pltpu.VMEM((N_PAD, 512), jnp.float32),
             pltpu.VMEM((N_PAD, 256), jnp.float32)]
            + [pltpu.SemaphoreType.DMA] * 10
        ),
    )
    return pl.pallas_call(
        _mlp1_body,
        grid_spec=spec,
        out_shape=jax.ShapeDtypeStruct((N_PAD, D), jnp.float32),
    )(sched, x_s, prev_s, W1pre, b1pre, W1int, b1int, W1out, b1out)


def _make_sc_kernels():
    info = plsc.get_sparse_core_info()
    nc, ns = info.num_cores, info.num_subcores
    nw = nc * ns
    tok_w = N // nw
    mesh = plsc.VectorSubcoreMesh(core_axis_name="c", subcore_axis_name="s")

    def _wid():
        return lax.axis_index("s") * nc + lax.axis_index("c")

    @functools.partial(
        pl.kernel, mesh=mesh,
        out_type=[jax.ShapeDtypeStruct((N_PAD, D), jnp.float32),
                  jax.ShapeDtypeStruct((N_PAD, D), jnp.float32)],
        scratch_types=[
            pltpu.VMEM((tok_w,), jnp.int32), pltpu.VMEM((tok_w,), jnp.int32),
            pltpu.VMEM((tok_w, D), jnp.float32), pltpu.VMEM((tok_w, D), jnp.float32),
            pltpu.SemaphoreType.DMA, pltpu.SemaphoreType.DMA,
            pltpu.SemaphoreType.DMA, pltpu.SemaphoreType.DMA,
        ],
    )
    def scatter_in(x0_hbm, x1_hbm, pos0_hbm, pos1_hbm, x0s_hbm, x1s_hbm,
                   idx0_v, idx1_v, r0_v, r1_v, s0, s1, s2, s3):
        base = _wid() * tok_w
        c0 = pltpu.async_copy(x0_hbm.at[pl.ds(base, tok_w)], r0_v, s0)
        c1 = pltpu.async_copy(x1_hbm.at[pl.ds(base, tok_w)], r1_v, s1)
        c2 = pltpu.async_copy(pos0_hbm.at[pl.ds(base, tok_w)], idx0_v, s2)
        c3 = pltpu.async_copy(pos1_hbm.at[pl.ds(base, tok_w)], idx1_v, s3)
        c0.wait()
        c2.wait()
        c4 = pltpu.async_copy(r0_v, x0s_hbm.at[idx0_v], s0)
        c1.wait()
        c3.wait()
        c5 = pltpu.async_copy(r1_v, x1s_hbm.at[idx1_v], s1)
        c4.wait()
        c5.wait()

    @functools.partial(
        pl.kernel, mesh=mesh,
        out_type=jax.ShapeDtypeStruct((N_PAD, D), jnp.float32),
        scratch_types=[
            pltpu.VMEM((tok_w,), jnp.int32), pltpu.VMEM((tok_w,), jnp.int32),
            pltpu.VMEM((tok_w, D), jnp.float32),
            pltpu.SemaphoreType.DMA, pltpu.SemaphoreType.DMA,
        ],
    )
    def permute(h0s_hbm, pos0_hbm, pos1_hbm, prevs_hbm, idx0_v, idx1_v, rows_v, s0, s1):
        base = _wid() * tok_w
        c0 = pltpu.async_copy(pos0_hbm.at[pl.ds(base, tok_w)], idx0_v, s0)
        c1 = pltpu.async_copy(pos1_hbm.at[pl.ds(base, tok_w)], idx1_v, s1)
        c0.wait()
        pltpu.async_copy(h0s_hbm.at[idx0_v], rows_v, s0).wait()
        c1.wait()
        pltpu.async_copy(rows_v, prevs_hbm.at[idx1_v], s1).wait()

    @functools.partial(
        pl.kernel, mesh=mesh,
        out_type=jax.ShapeDtypeStruct((N, D), jnp.float32),
        scratch_types=[
            pltpu.VMEM((tok_w,), jnp.int32),
            pltpu.VMEM((tok_w, D), jnp.float32),
            pltpu.SemaphoreType.DMA,
        ],
    )
    def gather_out(o1s_hbm, pos1_hbm, out_hbm, idx_v, rows_v, sem):
        base = _wid() * tok_w
        pltpu.sync_copy(pos1_hbm.at[pl.ds(base, tok_w)], idx_v)
        pltpu.async_copy(o1s_hbm.at[idx_v], rows_v, sem).wait()
        pltpu.sync_copy(rows_v, out_hbm.at[pl.ds(base, tok_w)])

    return scatter_in, permute, gather_out


def kernel(input_val, W0a, b0a, W0b, b0b, W0c, b0c, W1pre, b1pre, W1int, b1int, W1out, b1out):
    x0 = input_val[:, 0:256]
    x1 = input_val[:, 256:512]
    oh01 = input_val[:, 512:528]
    pos0, pos1, sched = _routing(oh01)

    scatter_in, permute, gather_out = _make_sc_kernels()

    x0_s, x1_s = scatter_in(x0, x1, pos0, pos1)
    h0_s = _tile_mlp0(sched, x0_s, W0a, b0a, W0b, b0b, W0c, b0c)
    prev_s = permute(h0_s, pos0, pos1)
    o1_s = _tile_mlp1(sched, x1_s, prev_s, W1pre, b1pre, W1int, b1int, W1out, b1out)
    return gather_out(o1_s, pos1)
